# Initial kernel scaffold; baseline (speedup 1.0000x reference)
import jax
import jax.numpy as jnp
from jax.experimental import pallas as pl


def kernel(mem, target, mask, keys, hit_flags):
    raise NotImplementedError("write your pallas kernel here")


# speedup vs baseline: 1.0143x; 1.0143x over previous
import jax
import jax.numpy as jnp
from jax.experimental import pallas as pl

_B = 4096
_D = 6
_BB = 512


def _stain_body(t_ref, m_ref, o_ref):
    t = t_ref[...]
    m = m_ref[...]
    denom = jnp.sum(m, axis=2) + 1e-6
    tm = t * m
    mean = jnp.sum(tm, axis=2) / denom
    var = jnp.sum(((t - mean[:, :, None]) ** 2) * m, axis=2) / denom
    std = jnp.sqrt(var + 1e-6)
    sm = jnp.concatenate(
        [mean[:, 0:1], std[:, 0:1], mean[:, 1:2], std[:, 1:2],
         mean[:, 2:3], std[:, 2:3]], axis=1)
    o_ref[...] = sm


def _stain_extract(target, mask):
    B, C, P = target.shape
    grid = (B // _BB,)
    return pl.pallas_call(
        _stain_body,
        grid=grid,
        in_specs=[
            pl.BlockSpec((_BB, C, P), lambda i: (i, 0, 0)),
            pl.BlockSpec((_BB, 1, P), lambda i: (i, 0, 0)),
        ],
        out_specs=pl.BlockSpec((_BB, _D), lambda i: (i, 0)),
        out_shape=jax.ShapeDtypeStruct((B, _D), jnp.float32),
    )(target, mask)


def kernel(mem, target, mask, keys, hit_flags):
    sm_miss = _stain_extract(target, mask)
    hit = hit_flags.astype(bool)
    sm_hit = jnp.take(mem, keys, axis=0)
    vals = jnp.where(hit[:, None], sm_hit, sm_miss)
    new_mem = mem.at[keys].set(vals)
    return jnp.take(new_mem, keys, axis=0)


# speedup vs baseline: 2.5133x; 2.4780x over previous
import jax
import jax.numpy as jnp
from jax.experimental import pallas as pl

_B = 4096
_D = 6
_BB = 512


def _stain_body(t_ref, m_ref, o_ref):
    t = t_ref[...]
    m = m_ref[...]
    denom = jnp.sum(m, axis=2) + 1e-6
    tm = t * m
    mean = jnp.sum(tm, axis=2) / denom
    var = jnp.sum(((t - mean[:, :, None]) ** 2) * m, axis=2) / denom
    std = jnp.sqrt(var + 1e-6)
    sm = jnp.concatenate(
        [mean[:, 0:1], std[:, 0:1], mean[:, 1:2], std[:, 1:2],
         mean[:, 2:3], std[:, 2:3]], axis=1)
    o_ref[...] = sm


def _stain_extract(target, mask):
    B, C, P = target.shape
    grid = (B // _BB,)
    return pl.pallas_call(
        _stain_body,
        grid=grid,
        in_specs=[
            pl.BlockSpec((_BB, C, P), lambda i: (i, 0, 0)),
            pl.BlockSpec((_BB, 1, P), lambda i: (i, 0, 0)),
        ],
        out_specs=pl.BlockSpec((_BB, _D), lambda i: (i, 0)),
        out_shape=jax.ShapeDtypeStruct((B, _D), jnp.float32),
    )(target, mask)


def kernel(mem, target, mask, keys, hit_flags):
    sm_miss = _stain_extract(target, mask)
    hit = hit_flags.astype(bool)
    sm_hit = jnp.take(mem, keys, axis=0)
    vals = jnp.where(hit[:, None], sm_hit, sm_miss)
    order = jnp.argsort(keys, stable=True)
    k_s = keys[order]
    last_pos = jnp.searchsorted(k_s, keys, side="right") - 1
    winner = order[last_pos]
    return vals[winner]
